# baseline (speedup 1.0000x reference)
import functools

import jax
import jax.numpy as jnp
from jax.experimental import pallas as pl
from jax.experimental.pallas import tpu as pltpu

N = 8000
TSTEPS = 199
REG_BLOCK_ROWS = 200


def _scan_body(a_ref, s_ref, pe_row, ne_row, kp_ref, out_ref, k_ref, st_ref):
    i = pl.program_id(0)

    @pl.when(i == 0)
    def _init():
        k_ref[...] = kp_ref[...]
        st_ref[0] = 0.0
        st_ref[1] = 1.0

    ai = a_ref[i]
    si = s_ref[i]
    active = st_ref[1] > 0.5
    cond = jnp.logical_and(active, si >= 0.0)

    k = k_ref[...]
    col = jax.lax.broadcasted_iota(jnp.int32, (1, N), 1)
    kval = jnp.sum(jnp.where(col == ai, k, 0.0))
    p = jnp.clip(kval, 0.01, 0.99)
    loss = st_ref[0]
    l_new = loss - (si * jnp.log(p) + (1.0 - si) * jnp.log(1.0 - p))
    k_new = jnp.clip(
        k + si * pe_row[...].reshape(1, N) + (1.0 - si) * ne_row[...].reshape(1, N),
        -30.0,
        30.0,
    )
    k_ref[...] = jnp.where(cond, k_new, k)
    st_ref[0] = jnp.where(cond, l_new, loss)
    st_ref[1] = jnp.where(cond, 1.0, 0.0)

    @pl.when(i == pl.num_programs(0) - 1)
    def _fini():
        out_ref[0] = st_ref[0]


_LN2 = 0.6931471805599453
_Q5 = (
    -0.023689253842601152,
    0.10028720550179689,
    -0.20866966038537965,
    0.3244118093469767,
    -0.4991878509816388,
    0.99998187218031,
)


def _fast_log1p_abs(x):
    y = jnp.abs(x) + 1.0
    bits = jax.lax.bitcast_convert_type(y, jnp.int32)
    e = jax.lax.shift_right_logical(bits, 23) - 127
    m_bits = jax.lax.bitwise_or(
        jax.lax.bitwise_and(bits, 0x007FFFFF), 0x3F800000
    )
    t = jax.lax.bitcast_convert_type(m_bits, jnp.float32) - 1.0
    q = _Q5[0]
    for c in _Q5[1:]:
        q = q * t + c
    return e.astype(jnp.float32) * _LN2 + t * q


def _reg_body(pe_blk, ne_blk, out_ref, acc_ref):
    i = pl.program_id(0)

    @pl.when(i == 0)
    def _init():
        acc_ref[0] = 0.0

    x = pe_blk[...]
    y = ne_blk[...]
    acc_ref[0] += jnp.sum(_fast_log1p_abs(x)) + jnp.sum(_fast_log1p_abs(y))

    @pl.when(i == pl.num_programs(0) - 1)
    def _fini():
        out_ref[0] = acc_ref[0]


def _loss_call(a, s, pe, ne, kp, interpret=False):
    grid_spec = pltpu.PrefetchScalarGridSpec(
        num_scalar_prefetch=2,
        grid=(TSTEPS,),
        in_specs=[
            pl.BlockSpec((1, 1, N), lambda i, a_ref, s_ref: (a_ref[i], 0, 0)),
            pl.BlockSpec((1, 1, N), lambda i, a_ref, s_ref: (a_ref[i], 0, 0)),
            pl.BlockSpec((1, N), lambda i, a_ref, s_ref: (0, 0)),
        ],
        out_specs=pl.BlockSpec(memory_space=pltpu.SMEM),
        scratch_shapes=[
            pltpu.VMEM((1, N), jnp.float32),
            pltpu.SMEM((2,), jnp.float32),
        ],
    )
    return pl.pallas_call(
        _scan_body,
        grid_spec=grid_spec,
        out_shape=jax.ShapeDtypeStruct((1,), jnp.float32),
        interpret=interpret,
    )(a, s, pe.reshape(N, 1, N), ne.reshape(N, 1, N), kp.reshape(1, N))[0]


def _reg_call(pe, ne, interpret=False):
    nblk = N // REG_BLOCK_ROWS
    return pl.pallas_call(
        _reg_body,
        grid=(nblk,),
        in_specs=[
            pl.BlockSpec((REG_BLOCK_ROWS, N), lambda i: (i, 0)),
            pl.BlockSpec((REG_BLOCK_ROWS, N), lambda i: (i, 0)),
        ],
        out_specs=pl.BlockSpec(memory_space=pltpu.SMEM),
        out_shape=jax.ShapeDtypeStruct((1,), jnp.float32),
        scratch_shapes=[pltpu.SMEM((1,), jnp.float32)],
        interpret=interpret,
    )(pe, ne)[0]


@functools.partial(jax.jit, static_argnames=("interpret",))
def _kernel_impl(a, s, pe, ne, kp, interpret=False):
    a32 = a[:TSTEPS].astype(jnp.int32)
    s32 = s[:TSTEPS].astype(jnp.float32)
    loss = _loss_call(a32, s32, pe, ne, kp, interpret=interpret)
    reg = _reg_call(pe, ne, interpret=interpret)
    return loss + 0.5 * reg


def kernel(a, s, pe, ne, kp):
    return _kernel_impl(a, s, pe, ne, kp)


# speedup vs baseline: 2.2125x; 2.2125x over previous
import functools

import jax
import jax.numpy as jnp
from jax.experimental import pallas as pl
from jax.experimental.pallas import tpu as pltpu
from jax.experimental.pallas import tpu_sc as plsc

N = 8000
NSTEP = 199
TPAD = 208
NGROUPS = 13
GCHUNKS = 26
REG_BLOCK_ROWS = 200
NCORES = 2


def _bcast_lane(vec, lane_idx):
    return jax.lax.gather(
        vec,
        jnp.full((16, 1), lane_idx, jnp.int32),
        jax.lax.GatherDimensionNumbers(
            offset_dims=(), collapsed_slice_dims=(0,), start_index_map=(0,)
        ),
        (1,),
        mode=jax.lax.GatherScatterMode.PROMISE_IN_BOUNDS,
    )


def _sc_scan_body(
    a_hbm,
    s_hbm,
    pe_hbm,
    ne_hbm,
    kp_hbm,
    p_out,
    act_out,
    a_v,
    s_v,
    idx_v,
    pg_v,
    qg_v,
    kpi_v,
    kk0_v,
    st_v,
    sem1,
    sem2,
):
    wid = jax.lax.axis_index("s") * NCORES + jax.lax.axis_index("c")

    @pl.when(wid < NGROUPS)
    def _work():
        pltpu.sync_copy(a_hbm, a_v)
        pltpu.sync_copy(s_hbm, s_v)
        acols = a_v[pl.ds(wid * 16, 16)]

        kpi_v[...] = acols
        pltpu.async_copy(kp_hbm.at[kpi_v], kk0_v, sem1).wait()

        lane = jax.lax.iota(jnp.int32, 16)

        def build(i, carry):
            ag = a_v[pl.ds((i // 16) * 16, 16)]
            ai_b = _bcast_lane(ag, i % 16)
            idx_v[i // 8, pl.ds((i % 8) * 16, 16)] = ai_b * N + acols
            return carry

        jax.lax.fori_loop(0, TPAD, build, 0)

        copies = []
        for c in range(GCHUNKS):
            copies.append(pltpu.async_copy(pe_hbm.at[idx_v.at[c]], pg_v.at[c], sem1))
            copies.append(pltpu.async_copy(ne_hbm.at[idx_v.at[c]], qg_v.at[c], sem2))
        for cp in copies:
            cp.wait()

        ids = lane + wid * 16
        kk = kk0_v[...]

        def step(i, carry):
            kk, pvec, avec, act = carry
            sg = s_v[pl.ds((i // 16) * 16, 16)]
            si = _bcast_lane(sg, i % 16)
            pi = pg_v[i // 8, pl.ds((i % 8) * 16, 16)]
            qi = qg_v[i // 8, pl.ds((i % 8) * 16, 16)]
            sge = jnp.where(si >= 0.0, 1.0, 0.0)
            condf = act * sge
            hitf = jnp.where(ids == i, 1.0, 0.0)
            recf = hitf * condf
            pvec = pvec + recf * (jnp.clip(kk, 0.01, 0.99) - pvec)
            avec = avec + recf * (1.0 - avec)
            kk_new = jnp.clip(kk + si * pi + (1.0 - si) * qi, -30.0, 30.0)
            kk = kk + condf * (kk_new - kk)
            return kk, pvec, avec, condf

        _, pvec, avec, _ = jax.lax.fori_loop(
            0,
            NSTEP,
            step,
            (
                kk,
                jnp.full((16,), 0.5, jnp.float32),
                jnp.zeros((16,), jnp.float32),
                jnp.ones((16,), jnp.float32),
            ),
        )

        st_v[...] = pvec
        pltpu.sync_copy(st_v, p_out.at[pl.ds(wid * 16, 16)])
        st_v[...] = avec
        pltpu.sync_copy(st_v, act_out.at[pl.ds(wid * 16, 16)])


def _sc_scan(a, s, pe_flat, ne_flat, kp, interpret=False):
    fn = pl.kernel(
        _sc_scan_body,
        out_type=[
            jax.ShapeDtypeStruct((TPAD,), jnp.float32),
            jax.ShapeDtypeStruct((TPAD,), jnp.float32),
        ],
        mesh=plsc.VectorSubcoreMesh(
            core_axis_name="c", subcore_axis_name="s", num_cores=2, num_subcores=16
        ),
        scratch_types=[
            pltpu.VMEM((TPAD,), jnp.int32),
            pltpu.VMEM((TPAD,), jnp.float32),
            pltpu.VMEM((GCHUNKS, 128), jnp.int32),
            pltpu.VMEM((GCHUNKS, 128), jnp.float32),
            pltpu.VMEM((GCHUNKS, 128), jnp.float32),
            pltpu.VMEM((16,), jnp.int32),
            pltpu.VMEM((16,), jnp.float32),
            pltpu.VMEM((16,), jnp.float32),
            pltpu.SemaphoreType.DMA,
            pltpu.SemaphoreType.DMA,
        ],
        interpret=interpret,
    )
    return fn(a, s, pe_flat, ne_flat, kp)


def _reg_body(pe_blk, ne_blk, p_ref, act_ref, s_ref, out_ref, acc_ref):
    i = pl.program_id(0)

    @pl.when(i == 0)
    def _init():
        acc_ref[0] = 0.0

    x = pe_blk[...]
    y = ne_blk[...]
    acc_ref[0] += jnp.sum(jnp.log(jnp.abs(x) + 1.0)) + jnp.sum(
        jnp.log(jnp.abs(y) + 1.0)
    )

    @pl.when(i == pl.num_programs(0) - 1)
    def _fini():
        p = p_ref[...]
        act = act_ref[...]
        s = s_ref[...]
        valid = jax.lax.broadcasted_iota(jnp.int32, (1, TPAD), 1) < NSTEP
        terms = act * (s * jnp.log(p) + (1.0 - s) * jnp.log(1.0 - p))
        loss = -jnp.sum(jnp.where(valid, terms, 0.0))
        out_ref[0] = loss + 0.5 * acc_ref[0]


def _reg_call(pe, ne, p, act, s, interpret=False):
    nblk = N // REG_BLOCK_ROWS
    return pl.pallas_call(
        _reg_body,
        grid=(nblk,),
        in_specs=[
            pl.BlockSpec((REG_BLOCK_ROWS, N), lambda i: (i, 0)),
            pl.BlockSpec((REG_BLOCK_ROWS, N), lambda i: (i, 0)),
            pl.BlockSpec((1, TPAD), lambda i: (0, 0)),
            pl.BlockSpec((1, TPAD), lambda i: (0, 0)),
            pl.BlockSpec((1, TPAD), lambda i: (0, 0)),
        ],
        out_specs=pl.BlockSpec(memory_space=pltpu.SMEM),
        out_shape=jax.ShapeDtypeStruct((1,), jnp.float32),
        scratch_shapes=[pltpu.SMEM((1,), jnp.float32)],
        interpret=interpret,
    )(pe, ne, p, act, s)[0]


@functools.partial(jax.jit, static_argnames=("interpret",))
def _kernel_impl(a, s, pe, ne, kp, interpret=False):
    a32 = jnp.zeros((TPAD,), jnp.int32).at[:NSTEP].set(a[:NSTEP].astype(jnp.int32))
    s32 = jnp.zeros((TPAD,), jnp.float32).at[:NSTEP].set(s[:NSTEP].astype(jnp.float32))
    p, act = _sc_scan(a32, s32, pe.reshape(-1), ne.reshape(-1), kp, interpret=interpret)
    return _reg_call(
        pe, ne, p.reshape(1, TPAD), act.reshape(1, TPAD), s32.reshape(1, TPAD),
        interpret=interpret,
    )


def kernel(a, s, pe, ne, kp):
    return _kernel_impl(a, s, pe, ne, kp)


# speedup vs baseline: 2.2326x; 1.0091x over previous
import functools

import jax
import jax.numpy as jnp
from jax.experimental import pallas as pl
from jax.experimental.pallas import tpu as pltpu
from jax.experimental.pallas import tpu_sc as plsc

N = 8000
NSTEP = 199
TPAD = 208
NGROUPS = 13
GCHUNKS = 26
REG_BLOCK_ROWS = 400
NCORES = 2


def _bcast_lane(vec, lane_idx):
    return jax.lax.gather(
        vec,
        jnp.full((16, 1), lane_idx, jnp.int32),
        jax.lax.GatherDimensionNumbers(
            offset_dims=(), collapsed_slice_dims=(0,), start_index_map=(0,)
        ),
        (1,),
        mode=jax.lax.GatherScatterMode.PROMISE_IN_BOUNDS,
    )


def _sc_scan_body(
    a_hbm,
    s_hbm,
    pe_hbm,
    ne_hbm,
    kp_hbm,
    p_out,
    act_out,
    a_v,
    s_v,
    idx_v,
    pg_v,
    qg_v,
    kpi_v,
    kk0_v,
    st_v,
    sem1,
    sem2,
):
    wid = jax.lax.axis_index("s") * NCORES + jax.lax.axis_index("c")

    @pl.when(wid < NGROUPS)
    def _work():
        pltpu.sync_copy(a_hbm, a_v)
        pltpu.sync_copy(s_hbm, s_v)
        acols = a_v[pl.ds(wid * 16, 16)]

        kpi_v[...] = acols
        pltpu.async_copy(kp_hbm.at[kpi_v], kk0_v, sem1).wait()

        lane = jax.lax.iota(jnp.int32, 16)

        def build(i, carry):
            ag = a_v[pl.ds((i // 16) * 16, 16)]
            ai_b = _bcast_lane(ag, i % 16)
            idx_v[i // 8, pl.ds((i % 8) * 16, 16)] = ai_b * N + acols
            return carry

        jax.lax.fori_loop(0, TPAD, build, 0)

        copies = []
        for c in range(GCHUNKS):
            copies.append(pltpu.async_copy(pe_hbm.at[idx_v.at[c]], pg_v.at[c], sem1))
            copies.append(pltpu.async_copy(ne_hbm.at[idx_v.at[c]], qg_v.at[c], sem2))
        for cp in copies:
            cp.wait()

        ids = lane + wid * 16
        kk = kk0_v[...]

        def step(i, carry):
            kk, pvec, avec, act = carry
            sg = s_v[pl.ds((i // 16) * 16, 16)]
            si = _bcast_lane(sg, i % 16)
            pi = pg_v[i // 8, pl.ds((i % 8) * 16, 16)]
            qi = qg_v[i // 8, pl.ds((i % 8) * 16, 16)]
            sge = jnp.where(si >= 0.0, 1.0, 0.0)
            condf = act * sge
            hitf = jnp.where(ids == i, 1.0, 0.0)
            recf = hitf * condf
            pvec = pvec + recf * (jnp.clip(kk, 0.01, 0.99) - pvec)
            avec = avec + recf * (1.0 - avec)
            kk_new = jnp.clip(kk + si * pi + (1.0 - si) * qi, -30.0, 30.0)
            kk = kk + condf * (kk_new - kk)
            return kk, pvec, avec, condf

        _, pvec, avec, _ = jax.lax.fori_loop(
            0,
            NSTEP,
            step,
            (
                kk,
                jnp.full((16,), 0.5, jnp.float32),
                jnp.zeros((16,), jnp.float32),
                jnp.ones((16,), jnp.float32),
            ),
        )

        st_v[...] = pvec
        pltpu.sync_copy(st_v, p_out.at[pl.ds(wid * 16, 16)])
        st_v[...] = avec
        pltpu.sync_copy(st_v, act_out.at[pl.ds(wid * 16, 16)])


def _sc_scan(a, s, pe_flat, ne_flat, kp, interpret=False):
    fn = pl.kernel(
        _sc_scan_body,
        out_type=[
            jax.ShapeDtypeStruct((TPAD,), jnp.float32),
            jax.ShapeDtypeStruct((TPAD,), jnp.float32),
        ],
        mesh=plsc.VectorSubcoreMesh(
            core_axis_name="c", subcore_axis_name="s", num_cores=2, num_subcores=16
        ),
        scratch_types=[
            pltpu.VMEM((TPAD,), jnp.int32),
            pltpu.VMEM((TPAD,), jnp.float32),
            pltpu.VMEM((GCHUNKS, 128), jnp.int32),
            pltpu.VMEM((GCHUNKS, 128), jnp.float32),
            pltpu.VMEM((GCHUNKS, 128), jnp.float32),
            pltpu.VMEM((16,), jnp.int32),
            pltpu.VMEM((16,), jnp.float32),
            pltpu.VMEM((16,), jnp.float32),
            pltpu.SemaphoreType.DMA,
            pltpu.SemaphoreType.DMA,
        ],
        interpret=interpret,
    )
    return fn(a, s, pe_flat, ne_flat, kp)


def _reg_body(pe_blk, ne_blk, p_ref, act_ref, s_ref, out_ref, acc_ref):
    i = pl.program_id(0)

    @pl.when(i == 0)
    def _init():
        acc_ref[0] = 0.0

    x = pe_blk[...]
    y = ne_blk[...]
    acc_ref[0] += jnp.sum(jnp.log(jnp.abs(x) + 1.0)) + jnp.sum(
        jnp.log(jnp.abs(y) + 1.0)
    )

    @pl.when(i == pl.num_programs(0) - 1)
    def _fini():
        p = p_ref[...]
        act = act_ref[...]
        s = s_ref[...]
        valid = jax.lax.broadcasted_iota(jnp.int32, (1, TPAD), 1) < NSTEP
        terms = act * (s * jnp.log(p) + (1.0 - s) * jnp.log(1.0 - p))
        loss = -jnp.sum(jnp.where(valid, terms, 0.0))
        out_ref[0] = loss + 0.5 * acc_ref[0]


def _reg_call(pe, ne, p, act, s, interpret=False):
    nblk = N // REG_BLOCK_ROWS
    return pl.pallas_call(
        _reg_body,
        grid=(nblk,),
        in_specs=[
            pl.BlockSpec((REG_BLOCK_ROWS, N), lambda i: (i, 0)),
            pl.BlockSpec((REG_BLOCK_ROWS, N), lambda i: (i, 0)),
            pl.BlockSpec((1, TPAD), lambda i: (0, 0)),
            pl.BlockSpec((1, TPAD), lambda i: (0, 0)),
            pl.BlockSpec((1, TPAD), lambda i: (0, 0)),
        ],
        out_specs=pl.BlockSpec(memory_space=pltpu.SMEM),
        out_shape=jax.ShapeDtypeStruct((1,), jnp.float32),
        scratch_shapes=[pltpu.SMEM((1,), jnp.float32)],
        interpret=interpret,
    )(pe, ne, p, act, s)[0]


@functools.partial(jax.jit, static_argnames=("interpret",))
def _kernel_impl(a, s, pe, ne, kp, interpret=False):
    a32 = jnp.zeros((TPAD,), jnp.int32).at[:NSTEP].set(a[:NSTEP].astype(jnp.int32))
    s32 = jnp.zeros((TPAD,), jnp.float32).at[:NSTEP].set(s[:NSTEP].astype(jnp.float32))
    p, act = _sc_scan(a32, s32, pe.reshape(-1), ne.reshape(-1), kp, interpret=interpret)
    return _reg_call(
        pe, ne, p.reshape(1, TPAD), act.reshape(1, TPAD), s32.reshape(1, TPAD),
        interpret=interpret,
    )


def kernel(a, s, pe, ne, kp):
    return _kernel_impl(a, s, pe, ne, kp)
